# baseline (speedup 1.0000x reference)
import functools

import jax
import jax.numpy as jnp
from jax import lax
from jax.experimental import pallas as pl
from jax.experimental.pallas import tpu as pltpu
from jax.experimental.pallas import tpu_sc as plsc

N_NODES = 10000
N_EDGES = 320000
F = 128

NC = 2
NS = 16
FH = F // NC
LANES = 16

K = 128
CHUNKS = 160
EDGES_PER_TILE = K * CHUNKS
N_EDGES_PAD = EDGES_PER_TILE * NS
ACC_ROWS = N_NODES + LANES
ROWS_PER_TILE = N_NODES // NS
ZROWS = 25

_mesh = plsc.VectorSubcoreMesh(core_axis_name="c", subcore_axis_name="s")


@functools.partial(
    pl.kernel,
    out_type=jax.ShapeDtypeStruct((N_NODES, F), jnp.float32),
    mesh=_mesh,
    scratch_types=[
        pltpu.VMEM((CHUNKS, K), jnp.int32),
        pltpu.VMEM((CHUNKS, K), jnp.int32),
        pltpu.VMEM((K, FH), jnp.float32),
        pltpu.VMEM((K, FH), jnp.float32),
        pltpu.VMEM((ZROWS, FH), jnp.float32),
        pltpu.VMEM_SHARED((ACC_ROWS, FH), jnp.float32),
        pltpu.SemaphoreType.DMA,
        pltpu.SemaphoreType.DMA,
    ],
    compiler_params=pltpu.CompilerParams(use_tc_tiling_on_sc=False),
)
def _sc_gather_scatter(feat_hbm, idx_hbm, out_hbm,
                       src_v, dst_v, rows_a, rows_b, zero_v, accum_sh,
                       sem_a, sem_b):
    c = lax.axis_index("c")
    s = lax.axis_index("s")

    pltpu.sync_copy(idx_hbm.at[s], src_v)
    pltpu.sync_copy(idx_hbm.at[NS + s], dst_v)

    def _xform_row(r):
        for l in range(K // LANES):
            v = src_v[r, pl.ds(l * LANES, LANES)]
            src_v[r, pl.ds(l * LANES, LANES)] = v * 2 + c

    def _zstore(q, carry):
        i = q // (FH // LANES)
        l = q % (FH // LANES)
        zero_v[i, pl.ds(l * LANES, LANES)] = jnp.zeros((LANES,), jnp.float32)
        return carry

    lax.fori_loop(0, ZROWS * (FH // LANES), _zstore, 0)

    def _zcopy(t, carry):
        pltpu.sync_copy(zero_v,
                        accum_sh.at[pl.ds(s * ROWS_PER_TILE + t * ZROWS, ZROWS)])
        return carry

    lax.fori_loop(0, ROWS_PER_TILE // ZROWS, _zcopy, 0)
    @pl.when(s == 0)
    def _():
        pltpu.sync_copy(zero_v.at[pl.ds(0, LANES)],
                        accum_sh.at[pl.ds(N_NODES, LANES)])
    plsc.subcore_barrier()

    _xform_row(0)
    _xform_row(1)
    pltpu.async_copy(feat_hbm.at[src_v.at[0]], rows_a, sem_a)
    pltpu.async_copy(feat_hbm.at[src_v.at[1]], rows_b, sem_b)

    def _pair(jj, carry):
        j = 2 * jj
        pltpu.make_async_copy(feat_hbm.at[src_v.at[j]], rows_a, sem_a).wait()
        pltpu.sync_copy(rows_a, accum_sh.at[dst_v.at[j]], add=True)
        _xform_row(j + 2)
        pltpu.async_copy(feat_hbm.at[src_v.at[j + 2]], rows_a, sem_a)
        pltpu.make_async_copy(feat_hbm.at[src_v.at[j + 1]], rows_b, sem_b).wait()
        pltpu.sync_copy(rows_b, accum_sh.at[dst_v.at[j + 1]], add=True)
        _xform_row(j + 3)
        pltpu.async_copy(feat_hbm.at[src_v.at[j + 3]], rows_b, sem_b)
        return carry

    lax.fori_loop(0, CHUNKS // 2 - 1, _pair, 0)
    pltpu.make_async_copy(feat_hbm.at[src_v.at[CHUNKS - 2]], rows_a, sem_a).wait()
    pltpu.sync_copy(rows_a, accum_sh.at[dst_v.at[CHUNKS - 2]], add=True)
    pltpu.make_async_copy(feat_hbm.at[src_v.at[CHUNKS - 1]], rows_b, sem_b).wait()
    pltpu.sync_copy(rows_b, accum_sh.at[dst_v.at[CHUNKS - 1]], add=True)
    plsc.subcore_barrier()

    pltpu.sync_copy(accum_sh.at[pl.ds(s * ROWS_PER_TILE, ROWS_PER_TILE)],
                    out_hbm.at[pl.ds(s * ROWS_PER_TILE, ROWS_PER_TILE),
                               pl.ds(c * FH, FH)])


def _tc_linear_body(h_ref, w_ref, b_ref, o_ref):
    o_ref[...] = lax.dot_general(
        h_ref[...], w_ref[...], (((1,), (1,)), ((), ())),
        preferred_element_type=jnp.float32) + b_ref[...]


_BM = 1000


@jax.jit
def _tc_linear(h, W, b2d):
    return pl.pallas_call(
        _tc_linear_body,
        grid=(N_NODES // _BM,),
        in_specs=[
            pl.BlockSpec((_BM, F), lambda i: (i, 0)),
            pl.BlockSpec((F, F), lambda i: (0, 0)),
            pl.BlockSpec((1, F), lambda i: (0, 0)),
        ],
        out_specs=pl.BlockSpec((_BM, F), lambda i: (i, 0)),
        out_shape=jax.ShapeDtypeStruct((N_NODES, F), jnp.float32),
    )(h, W, b2d)


def kernel(feature, edge_index, W, b):
    ei = edge_index.astype(jnp.int32)
    pad = jnp.tile(
        jnp.array([[0], [N_NODES]], dtype=jnp.int32), (1, N_EDGES_PAD - N_EDGES))
    idx = jnp.concatenate([ei, pad], axis=1)
    idx = idx.reshape(2 * NS, CHUNKS, K)
    feat_half = feature.reshape(2 * N_NODES, FH)
    h = _sc_gather_scatter(feat_half, idx)
    return _tc_linear(h, W, b.reshape(1, F))
